# baseline (speedup 1.0000x reference)
import functools

import jax
import jax.numpy as jnp
from jax import lax
from jax.experimental import pallas as pl
from jax.experimental.pallas import tpu as pltpu
from jax.experimental.pallas import tpu_sc as plsc

_NUM_CORES = 2
_NUM_SUBCORES = 16
_NUM_WORKERS = _NUM_CORES * _NUM_SUBCORES
_LANES = 16


@functools.lru_cache(maxsize=None)
def _build(B, T, V, D):
    N = B * T
    assert N % _NUM_WORKERS == 0
    rows_per_w = N // _NUM_WORKERS
    assert rows_per_w % T == 0
    seqs_per_w = rows_per_w // T
    lanes_per_row = D // _LANES

    mesh = plsc.VectorSubcoreMesh(core_axis_name="c", subcore_axis_name="s")

    @functools.partial(
        pl.kernel,
        out_type=jax.ShapeDtypeStruct((N, D), jnp.float32),
        mesh=mesh,
        compiler_params=pltpu.CompilerParams(use_tc_tiling_on_sc=False),
        scratch_types=[
            pltpu.VMEM((rows_per_w,), jnp.int32),
            pltpu.VMEM((T, D), jnp.float32),
            pltpu.VMEM((T, D), jnp.float32),
            pltpu.SemaphoreType.DMA,
        ],
    )
    def emb(x_hbm, tok_hbm, pos_hbm, out_hbm, idx_v, pos_v, rows_v, sem):
        wid = lax.axis_index("s") * _NUM_CORES + lax.axis_index("c")
        base = wid * rows_per_w
        pltpu.sync_copy(x_hbm.at[pl.ds(base, rows_per_w)], idx_v)
        pltpu.sync_copy(pos_hbm, pos_v)

        def seq_body(s, carry):
            pltpu.async_copy(
                tok_hbm.at[idx_v.at[pl.ds(s * T, T)]], rows_v, sem
            ).wait()

            def add_body(r, c2):
                for c in range(lanes_per_row):
                    sl = pl.ds(c * _LANES, _LANES)
                    plsc.addupdate(rows_v.at[r, sl], pos_v[r, sl])
                return c2

            lax.fori_loop(0, T, add_body, 0, unroll=2)
            pltpu.sync_copy(rows_v, out_hbm.at[pl.ds(base + s * T, T)])
            return carry

        lax.fori_loop(0, seqs_per_w, seq_body, 0)

    return emb


def kernel(x, token_table, pos_table):
    B, T = x.shape
    V, D = token_table.shape
    emb = _build(B, T, V, D)
    flat_idx = x.reshape(-1).astype(jnp.int32)
    out = emb(flat_idx, token_table, pos_table)
    return out.reshape(B, T, D)


# speedup vs baseline: 1.0558x; 1.0558x over previous
import functools

import jax
import jax.numpy as jnp
from jax import lax
from jax.experimental import pallas as pl
from jax.experimental.pallas import tpu as pltpu
from jax.experimental.pallas import tpu_sc as plsc

_NUM_CORES = 2
_NUM_SUBCORES = 16
_NUM_WORKERS = _NUM_CORES * _NUM_SUBCORES
_LANES = 16
_NBUF = 4


@functools.lru_cache(maxsize=None)
def _build(B, T, V, D):
    N = B * T
    assert N % _NUM_WORKERS == 0
    rows_per_w = N // _NUM_WORKERS
    assert rows_per_w % T == 0
    nchunks = rows_per_w // T
    assert nchunks % _NBUF == 0
    lanes_per_row = D // _LANES

    mesh = plsc.VectorSubcoreMesh(core_axis_name="c", subcore_axis_name="s")

    @functools.partial(
        pl.kernel,
        out_type=jax.ShapeDtypeStruct((N, D), jnp.float32),
        mesh=mesh,
        compiler_params=pltpu.CompilerParams(use_tc_tiling_on_sc=False),
        scratch_types=[
            pltpu.VMEM((rows_per_w,), jnp.int32),
            pltpu.VMEM((T, D), jnp.float32),
            *[pltpu.VMEM((T, D), jnp.float32)] * _NBUF,
            *[pltpu.SemaphoreType.DMA] * _NBUF,
            *[pltpu.SemaphoreType.DMA] * _NBUF,
        ],
    )
    def emb(x_hbm, tok_hbm, pos_hbm, out_hbm, idx_v, pos_v, *bufs):
        rows = bufs[:_NBUF]
        gsem = bufs[_NBUF:2 * _NBUF]
        ssem = bufs[2 * _NBUF:]
        wid = lax.axis_index("s") * _NUM_CORES + lax.axis_index("c")
        base = wid * rows_per_w
        pltpu.sync_copy(x_hbm.at[pl.ds(base, rows_per_w)], idx_v)
        pltpu.sync_copy(pos_hbm, pos_v)

        def gather_desc(t, b):
            return pltpu.make_async_copy(
                tok_hbm.at[idx_v.at[pl.ds(t * T, T)]], rows[b], gsem[b]
            )

        def scatter_desc(t, b):
            return pltpu.make_async_copy(
                rows[b], out_hbm.at[pl.ds(base + t * T, T)], ssem[b]
            )

        gather_desc(0, 0).start()

        def outer(i, carry):
            for b in range(_NBUF):
                t = i * _NBUF + b
                nb = (b + 1) % _NBUF

                @pl.when(t >= _NBUF - 1)
                def _():
                    scatter_desc(t - (_NBUF - 1), nb).wait()

                @pl.when(t + 1 < nchunks)
                def _():
                    gather_desc(t + 1, nb).start()

                gather_desc(t, b).wait()

                @plsc.parallel_loop(0, T, unroll=8)
                def _(r):
                    for c in range(lanes_per_row):
                        sl = pl.ds(c * _LANES, _LANES)
                        plsc.addupdate(rows[b].at[r, sl], pos_v[r, sl])

                scatter_desc(t, b).start()
            return carry

        lax.fori_loop(0, nchunks // _NBUF, outer, 0)
        for t in range(nchunks - _NBUF + 1, nchunks):
            scatter_desc(t, t % _NBUF).wait()

    return emb


def kernel(x, token_table, pos_table):
    B, T = x.shape
    V, D = token_table.shape
    emb = _build(B, T, V, D)
    flat_idx = x.reshape(-1).astype(jnp.int32)
    out = emb(flat_idx, token_table, pos_table)
    return out.reshape(B, T, D)
